# baseline (speedup 1.0000x reference)
import functools

import jax
import jax.numpy as jnp
from jax.experimental import pallas as pl
from jax.experimental.pallas import tpu as pltpu

TS = 512
HS = TS // 2
NSLOT = 2


def _adapter_body(xl_ref, xh_ref, dwl_ref, dwh_ref, db_ref, uw_ref,
                  o_hbm, o_buf, sem_o, *, B, S, C, D, SB, T):
    mi = pl.program_id(0)
    b = pl.program_id(1)
    s = pl.program_id(2)
    t = (mi * pl.num_programs(1) + b) * SB + s
    slot = t % NSLOT

    xl = xl_ref[0]
    xh = xh_ref[0]
    dwl = dwl_ref[0, 0]
    dwh = dwh_ref[0, 0]
    db = db_ref[0, 0, 0]
    uw = uw_ref[0, 0]

    z = (
        jnp.dot(xl, dwl, preferred_element_type=jnp.float32)
        + jnp.dot(xh, dwh, preferred_element_type=jnp.float32)
        + db[None, :]
    )
    z = z * jax.nn.sigmoid(z)

    @pl.when(t >= NSLOT)
    def _wait_slot():
        tp = t - NSLOT
        bp = tp // SB
        sp = tp % SB
        base = sp * TS
        pltpu.make_async_copy(
            o_buf.at[slot, pl.ds(0, HS), :],
            o_hbm.at[0, bp, pl.ds(base, HS), :],
            sem_o.at[slot, 0],
        ).wait()
        pltpu.make_async_copy(
            o_buf.at[slot, pl.ds(HS, HS), :],
            o_hbm.at[0, bp, pl.ds(base + HS, HS), :],
            sem_o.at[slot, 1],
        ).wait()

    o_buf[slot, pl.ds(0, HS), :] = jnp.dot(
        z[:HS], uw, preferred_element_type=jnp.float32
    )
    pltpu.make_async_copy(
        o_buf.at[slot, pl.ds(0, HS), :],
        o_hbm.at[0, b, pl.ds(s * TS, HS), :],
        sem_o.at[slot, 0],
    ).start()

    o_buf[slot, pl.ds(HS, HS), :] = jnp.dot(
        z[HS:], uw, preferred_element_type=jnp.float32
    )
    pltpu.make_async_copy(
        o_buf.at[slot, pl.ds(HS, HS), :],
        o_hbm.at[0, b, pl.ds(s * TS + HS, HS), :],
        sem_o.at[slot, 1],
    ).start()

    @pl.when(t == T - 1)
    def _drain():
        for tq in range(max(0, T - NSLOT), T):
            bq, sq = tq // SB, tq % SB
            for h in range(2):
                pltpu.make_async_copy(
                    o_buf.at[tq % NSLOT, pl.ds(h * HS, HS), :],
                    o_hbm.at[0, bq, pl.ds(sq * TS + h * HS, HS), :],
                    sem_o.at[tq % NSLOT, h],
                ).wait()


@jax.jit
def kernel(x, expert_index, down_w, down_b, up_w):
    B, S, C = x.shape
    M, N, _, D = down_w.shape
    CH = C // 2
    SB = S // TS
    T = M * B * SB

    idx = expert_index.astype(jnp.int32)
    m = jnp.arange(M)[:, None]
    bdw = down_w[m, idx]
    bdb = down_b[m, idx].reshape(M, B, 1, D)
    buw = up_w[m, idx]

    grid = (M, B, SB)

    out = pl.pallas_call(
        functools.partial(_adapter_body, B=B, S=S, C=C, D=D, SB=SB, T=T),
        grid=grid,
        in_specs=[
            pl.BlockSpec((1, TS, CH), lambda mm, b, s: (b, s, 0)),
            pl.BlockSpec((1, TS, CH), lambda mm, b, s: (b, s, 1)),
            pl.BlockSpec((1, 1, CH, D), lambda mm, b, s: (mm, b, 0, 0)),
            pl.BlockSpec((1, 1, CH, D), lambda mm, b, s: (mm, b, 1, 0)),
            pl.BlockSpec((1, 1, 1, D), lambda mm, b, s: (mm, b, 0, 0)),
            pl.BlockSpec((1, 1, D, C), lambda mm, b, s: (mm, b, 0, 0)),
        ],
        out_specs=pl.BlockSpec(memory_space=pltpu.MemorySpace.HBM),
        out_shape=jax.ShapeDtypeStruct((M, B, S, C), jnp.float32),
        scratch_shapes=[
            pltpu.VMEM((NSLOT, TS, C), jnp.float32),
            pltpu.SemaphoreType.DMA((NSLOT, 2)),
        ],
        compiler_params=pltpu.CompilerParams(
            dimension_semantics=("arbitrary", "arbitrary", "arbitrary"),
        ),
    )(x, x, bdw, bdw, bdb, buw)
    return out


# speedup vs baseline: 1.0692x; 1.0692x over previous
import functools

import jax
import jax.numpy as jnp
from jax.experimental import pallas as pl
from jax.experimental.pallas import tpu as pltpu

TS = 1024
HS = TS // 2
NSLOT = 4


def _adapter_body(xl_ref, xh_ref, dwl_ref, dwh_ref, db_ref, uw_ref,
                  o_hbm, o_buf, sem_o, *, B, S, C, D, SB, T):
    mi = pl.program_id(0)
    b = pl.program_id(1)
    s = pl.program_id(2)
    t = (mi * pl.num_programs(1) + b) * SB + s
    slot = t % NSLOT

    xl = xl_ref[0]
    xh = xh_ref[0]
    dwl = dwl_ref[0, 0]
    dwh = dwh_ref[0, 0]
    db = db_ref[0, 0, 0]
    uw = uw_ref[0, 0]

    z = (
        jnp.dot(xl, dwl, preferred_element_type=jnp.float32)
        + jnp.dot(xh, dwh, preferred_element_type=jnp.float32)
        + db[None, :]
    )
    z = z * jax.nn.sigmoid(z)

    @pl.when(t >= NSLOT)
    def _wait_slot():
        tp = t - NSLOT
        bp = tp // SB
        sp = tp % SB
        base = sp * TS
        pltpu.make_async_copy(
            o_buf.at[slot, pl.ds(0, HS), :],
            o_hbm.at[0, bp, pl.ds(base, HS), :],
            sem_o.at[slot, 0],
        ).wait()
        pltpu.make_async_copy(
            o_buf.at[slot, pl.ds(HS, HS), :],
            o_hbm.at[0, bp, pl.ds(base + HS, HS), :],
            sem_o.at[slot, 1],
        ).wait()

    o_buf[slot, pl.ds(0, HS), :] = jnp.dot(
        z[:HS], uw, preferred_element_type=jnp.float32
    )
    pltpu.make_async_copy(
        o_buf.at[slot, pl.ds(0, HS), :],
        o_hbm.at[0, b, pl.ds(s * TS, HS), :],
        sem_o.at[slot, 0],
    ).start()

    o_buf[slot, pl.ds(HS, HS), :] = jnp.dot(
        z[HS:], uw, preferred_element_type=jnp.float32
    )
    pltpu.make_async_copy(
        o_buf.at[slot, pl.ds(HS, HS), :],
        o_hbm.at[0, b, pl.ds(s * TS + HS, HS), :],
        sem_o.at[slot, 1],
    ).start()

    @pl.when(t == T - 1)
    def _drain():
        for tq in range(max(0, T - NSLOT), T):
            bq, sq = tq // SB, tq % SB
            for h in range(2):
                pltpu.make_async_copy(
                    o_buf.at[tq % NSLOT, pl.ds(h * HS, HS), :],
                    o_hbm.at[0, bq, pl.ds(sq * TS + h * HS, HS), :],
                    sem_o.at[tq % NSLOT, h],
                ).wait()


@jax.jit
def kernel(x, expert_index, down_w, down_b, up_w):
    B, S, C = x.shape
    M, N, _, D = down_w.shape
    CH = C // 2
    SB = S // TS
    T = M * B * SB

    idx = expert_index.astype(jnp.int32)
    m = jnp.arange(M)[:, None]
    bdw = down_w[m, idx]
    bdb = down_b[m, idx].reshape(M, B, 1, D)
    buw = up_w[m, idx]

    grid = (M, B, SB)

    out = pl.pallas_call(
        functools.partial(_adapter_body, B=B, S=S, C=C, D=D, SB=SB, T=T),
        grid=grid,
        in_specs=[
            pl.BlockSpec((1, TS, CH), lambda mm, b, s: (b, s, 0)),
            pl.BlockSpec((1, TS, CH), lambda mm, b, s: (b, s, 1)),
            pl.BlockSpec((1, 1, CH, D), lambda mm, b, s: (mm, b, 0, 0)),
            pl.BlockSpec((1, 1, CH, D), lambda mm, b, s: (mm, b, 1, 0)),
            pl.BlockSpec((1, 1, 1, D), lambda mm, b, s: (mm, b, 0, 0)),
            pl.BlockSpec((1, 1, D, C), lambda mm, b, s: (mm, b, 0, 0)),
        ],
        out_specs=pl.BlockSpec(memory_space=pltpu.MemorySpace.HBM),
        out_shape=jax.ShapeDtypeStruct((M, B, S, C), jnp.float32),
        scratch_shapes=[
            pltpu.VMEM((NSLOT, TS, C), jnp.float32),
            pltpu.SemaphoreType.DMA((NSLOT, 2)),
        ],
        compiler_params=pltpu.CompilerParams(
            dimension_semantics=("arbitrary", "arbitrary", "arbitrary"),
        ),
    )(x, x, bdw, bdw, bdb, buw)
    return out
